# baseline (speedup 1.0000x reference)
import functools

import jax
import jax.numpy as jnp
from jax import lax
from jax.experimental import pallas as pl
from jax.experimental.pallas import tpu as pltpu
from jax.experimental.pallas import tpu_sc as plsc

_LANES = 16
_Q = 4


@functools.lru_cache(maxsize=None)
def _build(B, S, E, V):
    info = plsc.get_sparse_core_info()
    nw = info.num_cores * info.num_subcores
    assert E % _LANES == 0
    bq = B // _Q
    n_units = S * _Q // nw
    assert S * _Q % nw == 0 and bq % 128 == 0
    su = nw // _Q
    e_vecs = E // _LANES
    bqp = bq + 1
    chunks = [(off, 128) for off in range(0, bq, 128)]

    mesh = plsc.VectorSubcoreMesh(core_axis_name="c", subcore_axis_name="s")

    @functools.partial(
        pl.kernel,
        mesh=mesh,
        out_type=jax.ShapeDtypeStruct((S, E, B), jnp.float32),
        scratch_types=[
            pltpu.VMEM((2, bq), jnp.int32),
            pltpu.VMEM((2, bq, E), jnp.float32),
            pltpu.VMEM((2, E, bqp), jnp.float32),
            pltpu.VMEM((S, E), jnp.float32),
            pltpu.SemaphoreType.DMA,
            pltpu.SemaphoreType.DMA,
            pltpu.SemaphoreType.DMA,
            pltpu.SemaphoreType.DMA,
            pltpu.SemaphoreType.DMA,
            pltpu.SemaphoreType.DMA,
        ],
        compiler_params=pltpu.CompilerParams(
            use_tc_tiling_on_sc=False, needs_layout_passes=False),
    )
    def k(xt_hbm, tok_hbm, pos_hbm, out_hbm, idx_v, g_v, t_v, pos_v,
          si0, si1, sg0, sg1, ss0, ss1):
        wid = lax.axis_index("s") * info.num_cores + lax.axis_index("c")
        q = lax.rem(wid, _Q)
        s_base = lax.div(wid, _Q)
        qb = q * bq
        sem_i = (si0, si1)
        sem_g = (sg0, sg1)
        sem_s = (ss0, ss1)

        pltpu.sync_copy(pos_hbm, pos_v)

        def s_of(t):
            return s_base + su * t

        def fetch_idx(t, u):
            pltpu.async_copy(
                xt_hbm.at[s_of(t)].at[pl.ds(qb, bq)], idx_v.at[u], sem_i[u])

        def wait_idx(u):
            pltpu.make_async_copy(
                xt_hbm.at[0].at[pl.ds(0, bq)], idx_v.at[u], sem_i[u]).wait()

        def fetch_g(u):
            for off, sz in chunks:
                pltpu.async_copy(
                    tok_hbm.at[idx_v.at[u].at[pl.ds(off, sz)]],
                    g_v.at[u].at[pl.ds(off, sz)],
                    sem_g[u])

        def wait_g(u):
            pltpu.make_async_copy(
                tok_hbm.at[pl.ds(0, bq)], g_v.at[u], sem_g[u]).wait()

        def store(t, u):
            s = s_of(t)
            for e in range(E):
                pltpu.async_copy(
                    t_v.at[u].at[e].at[pl.ds(0, bq)],
                    out_hbm.at[s].at[e].at[pl.ds(qb, bq)],
                    sem_s[u])

        def wait_s(u):
            pltpu.make_async_copy(
                out_hbm.at[0].at[:, pl.ds(0, bq)],
                t_v.at[u].at[:, pl.ds(0, bq)], sem_s[u]).wait()

        iota = lax.iota(jnp.int32, _LANES)

        def combine(t, u):
            s = s_of(t)
            pvs = [pos_v[s, pl.ds(j * _LANES, _LANES)] for j in range(e_vecs)]
            ejs = [j * _LANES + iota for j in range(e_vecs)]

            def body(r, carry):
                pv = carry
                for j in range(e_vecs):
                    v = g_v[u, r, pl.ds(j * _LANES, _LANES)] + pv[j]
                    plsc.store_scatter(
                        t_v.at[u], [ejs[j], jnp.broadcast_to(r, (_LANES,))], v)
                return pv
            lax.fori_loop(0, bq, body, tuple(pvs))

        def unit(t, u, pre_g, pre_i, w_s):
            if pre_g:
                wait_idx(1 - u)
                fetch_g(1 - u)
            wait_g(u)
            if pre_i:
                fetch_idx(t + 2, u)
            if w_s:
                wait_s(u)
            combine(t, u)
            store(t, u)

        assert n_units >= 5 and n_units % 2 == 1
        fetch_idx(0, 0)
        wait_idx(0)
        fetch_g(0)
        fetch_idx(1, 1)

        unit(0, 0, True, True, False)
        unit(1, 1, True, True, False)

        def group(g2, _):
            for uu in (0, 1):
                unit(2 + 2 * g2 + uu, uu, True, True, True)
            return 0

        lax.fori_loop(0, (n_units - 5) // 2, group, 0)

        unit(n_units - 3, 0, True, True, True)
        unit(n_units - 2, 1, True, False, True)
        unit(n_units - 1, 0, False, False, True)
        wait_s(1)
        wait_s(0)

    return k


def kernel(x, token_table, pos_table):
    B, S = x.shape
    V, E = token_table.shape
    k = _build(B, S, E, V)
    xt = x.astype(jnp.int32).T
    out_t = k(xt, token_table, pos_table)
    return out_t.transpose(2, 0, 1)
